# baseline (speedup 1.0000x reference)
import numpy as np

import jax
import jax.numpy as jnp
from jax import lax
from jax.experimental import pallas as pl
from jax.experimental.pallas import tpu as pltpu

_TINY = np.float32(np.finfo(np.float32).tiny)


def _rotl(x, r):
    return lax.shift_left(x, jnp.uint32(r)) | lax.shift_right_logical(
        x, jnp.uint32(32 - r))


def _threefry2x32(k1, k2, x0, x1):
    ks = (k1, k2, k1 ^ k2 ^ jnp.uint32(0x1BD11BDA))
    x0 = x0 + ks[0]
    x1 = x1 + ks[1]
    rots = ((13, 15, 26, 6), (17, 29, 16, 24))
    sched = ((0, 1, 2), (1, 2, 0), (0, 0, 1), (1, 1, 2), (0, 2, 0))
    for i, (rset, ka, kb) in enumerate(sched):
        for r in rots[rset]:
            x0 = x0 + x1
            x1 = _rotl(x1, r)
            x1 = x0 ^ x1
        x0 = x0 + ks[ka]
        x1 = x1 + ks[kb] + jnp.uint32(i + 1)
    return x0, x1


def _sample_embed_kernel(key_ref, logits_ref, embeds_ref, out_ref):
    bi = pl.program_id(0)
    ti = pl.program_id(1)
    _, d, tb = logits_ref.shape
    t_total = tb * pl.num_programs(1)

    d_iota = lax.broadcasted_iota(jnp.uint32, (d, tb), 0)
    t_iota = lax.broadcasted_iota(jnp.uint32, (d, tb), 1)
    base = (lax.convert_element_type(bi, jnp.uint32) * jnp.uint32(d * t_total)
            + lax.convert_element_type(ti, jnp.uint32) * jnp.uint32(tb))
    cnt = base + d_iota * jnp.uint32(t_total) + t_iota

    o0, o1 = _threefry2x32(key_ref[0], key_ref[1], jnp.zeros_like(cnt), cnt)
    bits = o0 ^ o1

    fb = lax.shift_right_logical(bits, jnp.uint32(9)) | jnp.uint32(0x3F800000)
    f = lax.bitcast_convert_type(fb, jnp.float32) - jnp.float32(1.0)
    u = jnp.maximum(jnp.float32(_TINY), f + jnp.float32(_TINY))
    z = logits_ref[0] + (-jnp.log(-jnp.log(u)))

    zmax = jnp.max(z, axis=0, keepdims=True)
    idx = lax.broadcasted_iota(jnp.int32, (d, tb), 0)
    winner = jnp.min(jnp.where(z == zmax, idx, d), axis=0, keepdims=True)
    onehot = (idx == winner).astype(jnp.float32)

    out_ref[0] = lax.dot_general(
        embeds_ref[...], onehot, (((0,), (0,)), ((), ())),
        preferred_element_type=jnp.float32)


def kernel(logits, key, embeds):
    b, d, t = logits.shape
    e = embeds.shape[1]
    t_blk = 512

    grid_spec = pltpu.PrefetchScalarGridSpec(
        num_scalar_prefetch=1,
        grid=(b, t // t_blk),
        in_specs=[pl.BlockSpec((1, d, t_blk), lambda bi, ti, key: (bi, 0, ti)),
                  pl.BlockSpec((d, e), lambda bi, ti, key: (0, 0))],
        out_specs=pl.BlockSpec((1, e, t_blk), lambda bi, ti, key: (bi, 0, ti)),
    )
    out = pl.pallas_call(
        _sample_embed_kernel,
        out_shape=jax.ShapeDtypeStruct((b, e, t), jnp.float32),
        grid_spec=grid_spec,
        compiler_params=pltpu.CompilerParams(
            dimension_semantics=("parallel", "parallel"),
            vmem_limit_bytes=64 << 20),
    )(key, logits, embeds)
    return logits, out


# speedup vs baseline: 1.5111x; 1.5111x over previous
import numpy as np

import jax
import jax.numpy as jnp
from jax import lax
from jax.experimental import pallas as pl
from jax.experimental.pallas import tpu as pltpu

_TINY = np.float32(np.finfo(np.float32).tiny)
_CHUNK = 8


def _rotl(x, r):
    return lax.shift_left(x, jnp.uint32(r)) | lax.shift_right_logical(
        x, jnp.uint32(32 - r))


def _threefry2x32(k1, k2, x0, x1):
    ks = (k1, k2, k1 ^ k2 ^ jnp.uint32(0x1BD11BDA))
    rots = ((13, 15, 26, 6), (17, 29, 16, 24))
    sched = ((0, 1, 2), (1, 2, 0), (0, 0, 1), (1, 1, 2), (0, 2, 0))
    for i, (rset, ka, kb) in enumerate(sched):
        for r in rots[rset]:
            x0 = x0 + x1
            x1 = _rotl(x1, r)
            x1 = x0 ^ x1
        x0 = x0 + ks[ka]
        x1 = x1 + ks[kb] + jnp.uint32(i + 1)
    return x0, x1


def _sample_embed_kernel(key_ref, logits_ref, embeds_ref, out_ref, z_ref):
    bi = pl.program_id(0)
    ti = pl.program_id(1)
    _, d, tb = logits_ref.shape
    t_total = tb * pl.num_programs(1)
    k1 = key_ref[0]
    k2 = key_ref[1]

    d_iota = lax.broadcasted_iota(jnp.uint32, (_CHUNK, tb), 0)
    t_iota = lax.broadcasted_iota(jnp.uint32, (_CHUNK, tb), 1)
    inv = d_iota * jnp.uint32(t_total) + t_iota
    base = (lax.convert_element_type(bi, jnp.uint32) * jnp.uint32(d * t_total)
            + lax.convert_element_type(ti, jnp.uint32) * jnp.uint32(tb) + k2)

    zmax8 = None
    for c in range(d // _CHUNK):
        x1 = inv + (base + jnp.uint32(c * _CHUNK * t_total))
        o0, o1 = _threefry2x32(k1, k2, k1 + jnp.zeros_like(x1), x1)
        bits = o0 ^ o1
        fb = (lax.shift_right_logical(bits, jnp.uint32(9))
              | jnp.uint32(0x3F800000))
        f = lax.bitcast_convert_type(fb, jnp.float32) - jnp.float32(1.0)
        u = jnp.maximum(jnp.float32(_TINY), f + jnp.float32(_TINY))
        zc = logits_ref[0, pl.ds(c * _CHUNK, _CHUNK), :] - jnp.log(-jnp.log(u))
        z_ref[pl.ds(c * _CHUNK, _CHUNK), :] = zc
        zmax8 = zc if zmax8 is None else jnp.maximum(zmax8, zc)

    zmax = jnp.max(zmax8, axis=0, keepdims=True)

    idx8 = lax.broadcasted_iota(jnp.int32, (_CHUNK, tb), 0)
    winner8 = None
    for c in range(d // _CHUNK):
        zc = z_ref[pl.ds(c * _CHUNK, _CHUNK), :]
        cand = jnp.where(zc == zmax, idx8 + jnp.int32(c * _CHUNK), d)
        winner8 = cand if winner8 is None else jnp.minimum(winner8, cand)
    winner = jnp.min(winner8, axis=0, keepdims=True)

    idx = lax.broadcasted_iota(jnp.int32, (d, tb), 0)
    onehot = (idx == winner).astype(jnp.float32)

    out_ref[0] = lax.dot_general(
        embeds_ref[...], onehot, (((0,), (0,)), ((), ())),
        preferred_element_type=jnp.float32)


def kernel(logits, key, embeds):
    b, d, t = logits.shape
    e = embeds.shape[1]
    t_blk = 512

    grid_spec = pltpu.PrefetchScalarGridSpec(
        num_scalar_prefetch=1,
        grid=(b, t // t_blk),
        in_specs=[pl.BlockSpec((1, d, t_blk), lambda bi, ti, key: (bi, 0, ti)),
                  pl.BlockSpec((d, e), lambda bi, ti, key: (0, 0))],
        out_specs=pl.BlockSpec((1, e, t_blk), lambda bi, ti, key: (bi, 0, ti)),
        scratch_shapes=[pltpu.VMEM((d, t_blk), jnp.float32)],
    )
    out = pl.pallas_call(
        _sample_embed_kernel,
        out_shape=jax.ShapeDtypeStruct((b, e, t), jnp.float32),
        grid_spec=grid_spec,
        compiler_params=pltpu.CompilerParams(
            dimension_semantics=("parallel", "parallel"),
            vmem_limit_bytes=64 << 20),
    )(key, logits, embeds)
    return logits, out


# speedup vs baseline: 1.5267x; 1.0104x over previous
import numpy as np

import jax
import jax.numpy as jnp
from jax import lax
from jax.experimental import pallas as pl
from jax.experimental.pallas import tpu as pltpu

_TINY = np.float32(np.finfo(np.float32).tiny)
_CHUNK = 8


def _rotl(x, r):
    return lax.shift_left(x, jnp.uint32(r)) | lax.shift_right_logical(
        x, jnp.uint32(32 - r))


def _threefry2x32(k1, k2, x0, x1):
    ks = (k1, k2, k1 ^ k2 ^ jnp.uint32(0x1BD11BDA))
    rots = ((13, 15, 26, 6), (17, 29, 16, 24))
    sched = ((0, 1, 2), (1, 2, 0), (0, 0, 1), (1, 1, 2), (0, 2, 0))
    for i, (rset, ka, kb) in enumerate(sched):
        for r in rots[rset]:
            x0 = x0 + x1
            x1 = _rotl(x1, r)
            x1 = x0 ^ x1
        x0 = x0 + ks[ka]
        x1 = x1 + ks[kb] + jnp.uint32(i + 1)
    return x0, x1


def _sample_embed_kernel(key_ref, logits_ref, embeds_ref, out_ref):
    bi = pl.program_id(0)
    ti = pl.program_id(1)
    _, d, tb = logits_ref.shape
    t_total = tb * pl.num_programs(1)
    k1 = key_ref[0]
    k2 = key_ref[1]

    d_iota = lax.broadcasted_iota(jnp.uint32, (_CHUNK, tb), 0)
    t_iota = lax.broadcasted_iota(jnp.uint32, (_CHUNK, tb), 1)
    inv = d_iota * jnp.uint32(t_total) + t_iota
    base = (lax.convert_element_type(bi, jnp.uint32) * jnp.uint32(d * t_total)
            + lax.convert_element_type(ti, jnp.uint32) * jnp.uint32(tb) + k2)

    idx8 = lax.broadcasted_iota(jnp.int32, (_CHUNK, tb), 0)
    m8 = None
    w8 = None
    for c in range(d // _CHUNK):
        x1 = inv + (base + jnp.uint32(c * _CHUNK * t_total))
        o0, o1 = _threefry2x32(k1, k2, k1 + jnp.zeros_like(x1), x1)
        bits = o0 ^ o1
        fb = (lax.shift_right_logical(bits, jnp.uint32(9))
              | jnp.uint32(0x3F800000))
        f = lax.bitcast_convert_type(fb, jnp.float32) - jnp.float32(1.0)
        u = jnp.maximum(jnp.float32(_TINY), f + jnp.float32(_TINY))
        zc = logits_ref[0, pl.ds(c * _CHUNK, _CHUNK), :] - jnp.log(-jnp.log(u))
        if m8 is None:
            m8, w8 = zc, idx8
        else:
            better = zc > m8
            w8 = jnp.where(better, idx8 + jnp.int32(c * _CHUNK), w8)
            m8 = jnp.maximum(m8, zc)

    zmax = jnp.max(m8, axis=0, keepdims=True)
    winner = jnp.min(jnp.where(m8 == zmax, w8, d), axis=0, keepdims=True)

    idx = lax.broadcasted_iota(jnp.int32, (d, tb), 0)
    onehot = (idx == winner).astype(jnp.float32)

    out_ref[0] = lax.dot_general(
        embeds_ref[...], onehot, (((0,), (0,)), ((), ())),
        preferred_element_type=jnp.float32)


def kernel(logits, key, embeds):
    b, d, t = logits.shape
    e = embeds.shape[1]
    t_blk = 512

    grid_spec = pltpu.PrefetchScalarGridSpec(
        num_scalar_prefetch=1,
        grid=(b, t // t_blk),
        in_specs=[pl.BlockSpec((1, d, t_blk), lambda bi, ti, key: (bi, 0, ti)),
                  pl.BlockSpec((d, e), lambda bi, ti, key: (0, 0))],
        out_specs=pl.BlockSpec((1, e, t_blk), lambda bi, ti, key: (bi, 0, ti)),
    )
    out = pl.pallas_call(
        _sample_embed_kernel,
        out_shape=jax.ShapeDtypeStruct((b, e, t), jnp.float32),
        grid_spec=grid_spec,
        compiler_params=pltpu.CompilerParams(
            dimension_semantics=("parallel", "parallel"),
            vmem_limit_bytes=64 << 20),
    )(key, logits, embeds)
    return logits, out


# speedup vs baseline: 1.5881x; 1.0402x over previous
import numpy as np

import jax
import jax.numpy as jnp
from jax import lax
from jax.experimental import pallas as pl
from jax.experimental.pallas import tpu as pltpu

_TINY = np.float32(np.finfo(np.float32).tiny)
_CHUNK = 8


def _rotl(x, r):
    return lax.shift_left(x, jnp.uint32(r)) | lax.shift_right_logical(
        x, jnp.uint32(32 - r))


def _threefry2x32(k1, k2, x0, x1):
    ks = (k1, k2, k1 ^ k2 ^ jnp.uint32(0x1BD11BDA))
    rots = ((13, 15, 26, 6), (17, 29, 16, 24))
    sched = ((0, 1, 2), (1, 2, 0), (0, 0, 1), (1, 1, 2), (0, 2, 0))
    for i, (rset, ka, kb) in enumerate(sched):
        for r in rots[rset]:
            x0 = x0 + x1
            x1 = _rotl(x1, r)
            x1 = x0 ^ x1
        x0 = x0 + ks[ka]
        x1 = x1 + ks[kb] + jnp.uint32(i + 1)
    return x0, x1


def _sample_embed_kernel(key_ref, logits_ref, embeds_ref, out_ref):
    bi = pl.program_id(0)
    ti = pl.program_id(1)
    _, d, tb = logits_ref.shape
    t_total = tb * pl.num_programs(1)
    k1 = key_ref[0]
    k2 = key_ref[1]

    d_iota = lax.broadcasted_iota(jnp.uint32, (_CHUNK, tb), 0)
    t_iota = lax.broadcasted_iota(jnp.uint32, (_CHUNK, tb), 1)
    inv = d_iota * jnp.uint32(t_total) + t_iota
    base = (lax.convert_element_type(bi, jnp.uint32) * jnp.uint32(d * t_total)
            + lax.convert_element_type(ti, jnp.uint32) * jnp.uint32(tb) + k2)

    idx8 = lax.broadcasted_iota(jnp.int32, (_CHUNK, tb), 0)
    m8 = None
    w8 = None
    for c in range(d // _CHUNK):
        x1 = inv + (base + jnp.uint32(c * _CHUNK * t_total))
        o0, o1 = _threefry2x32(k1, k2, k1 + jnp.zeros_like(x1), x1)
        bits = o0 ^ o1
        fb = (lax.shift_right_logical(bits, jnp.uint32(9))
              | jnp.uint32(0x3F800000))
        f = lax.bitcast_convert_type(fb, jnp.float32) - jnp.float32(1.0)
        u = jnp.maximum(jnp.float32(_TINY), f + jnp.float32(_TINY))
        zc = logits_ref[0, pl.ds(c * _CHUNK, _CHUNK), :] - jnp.log(-jnp.log(u))
        if m8 is None:
            m8, w8 = zc, idx8
        else:
            better = zc > m8
            w8 = jnp.where(better, idx8 + jnp.int32(c * _CHUNK), w8)
            m8 = jnp.maximum(m8, zc)

    zmax = jnp.max(m8, axis=0, keepdims=True)
    winner = jnp.min(jnp.where(m8 == zmax, w8, d), axis=0, keepdims=True)

    idx = lax.broadcasted_iota(jnp.int32, (d, tb), 0)
    onehot = (idx == winner).astype(jnp.float32)

    out_ref[0] = lax.dot_general(
        embeds_ref[...], onehot, (((0,), (0,)), ((), ())),
        preferred_element_type=jnp.float32)


def kernel(logits, key, embeds):
    b, d, t = logits.shape
    e = embeds.shape[1]
    t_blk = 1024

    grid_spec = pltpu.PrefetchScalarGridSpec(
        num_scalar_prefetch=1,
        grid=(b, t // t_blk),
        in_specs=[pl.BlockSpec((1, d, t_blk), lambda bi, ti, key: (bi, 0, ti)),
                  pl.BlockSpec((d, e), lambda bi, ti, key: (0, 0))],
        out_specs=pl.BlockSpec((1, e, t_blk), lambda bi, ti, key: (bi, 0, ti)),
    )
    out = pl.pallas_call(
        _sample_embed_kernel,
        out_shape=jax.ShapeDtypeStruct((b, e, t), jnp.float32),
        grid_spec=grid_spec,
        compiler_params=pltpu.CompilerParams(
            dimension_semantics=("parallel", "parallel"),
            vmem_limit_bytes=64 << 20),
    )(key, logits, embeds)
    return logits, out


# speedup vs baseline: 1.6300x; 1.0264x over previous
import numpy as np

import jax
import jax.numpy as jnp
from jax import lax
from jax.experimental import pallas as pl
from jax.experimental.pallas import tpu as pltpu

_TINY = np.float32(np.finfo(np.float32).tiny)
_CHUNK = 8


def _rotl(x, r):
    return lax.shift_left(x, jnp.uint32(r)) | lax.shift_right_logical(
        x, jnp.uint32(32 - r))


def _threefry2x32(k1, k2, x0, x1):
    ks = (k1, k2, k1 ^ k2 ^ jnp.uint32(0x1BD11BDA))
    rots = ((13, 15, 26, 6), (17, 29, 16, 24))
    sched = ((0, 1, 2), (1, 2, 0), (0, 0, 1), (1, 1, 2), (0, 2, 0))
    for i, (rset, ka, kb) in enumerate(sched):
        for r in rots[rset]:
            x0 = x0 + x1
            x1 = _rotl(x1, r)
            x1 = x0 ^ x1
        x0 = x0 + ks[ka]
        x1 = x1 + ks[kb] + jnp.uint32(i + 1)
    return x0, x1


def _sample_embed_kernel(key_ref, logits_ref, embeds_ref, out_ref):
    bi = pl.program_id(0)
    ti = pl.program_id(1)
    _, d, tb = logits_ref.shape
    t_total = tb * pl.num_programs(1)
    k1 = key_ref[0]
    k2 = key_ref[1]

    d_iota = lax.broadcasted_iota(jnp.uint32, (_CHUNK, tb), 0)
    t_iota = lax.broadcasted_iota(jnp.uint32, (_CHUNK, tb), 1)
    inv = d_iota * jnp.uint32(t_total) + t_iota
    base = (lax.convert_element_type(bi, jnp.uint32) * jnp.uint32(d * t_total)
            + lax.convert_element_type(ti, jnp.uint32) * jnp.uint32(tb) + k2)

    idx8 = lax.broadcasted_iota(jnp.int32, (_CHUNK, tb), 0)
    m8 = None
    w8 = None
    for c in range(d // _CHUNK):
        x1 = inv + (base + jnp.uint32(c * _CHUNK * t_total))
        o0, o1 = _threefry2x32(k1, k2, k1 + jnp.zeros_like(x1), x1)
        bits = o0 ^ o1
        fb = (lax.shift_right_logical(bits, jnp.uint32(9))
              | jnp.uint32(0x3F800000))
        f = lax.bitcast_convert_type(fb, jnp.float32) - jnp.float32(1.0)
        u = jnp.maximum(jnp.float32(_TINY), f + jnp.float32(_TINY))
        zc = logits_ref[0, pl.ds(c * _CHUNK, _CHUNK), :] - jnp.log(-jnp.log(u))
        if m8 is None:
            m8, w8 = zc, idx8
        else:
            better = zc > m8
            w8 = jnp.where(better, idx8 + jnp.int32(c * _CHUNK), w8)
            m8 = jnp.maximum(m8, zc)

    zmax = jnp.max(m8, axis=0, keepdims=True)
    winner = jnp.min(jnp.where(m8 == zmax, w8, d), axis=0, keepdims=True)

    idx = lax.broadcasted_iota(jnp.int32, (d, tb), 0)
    onehot = (idx == winner).astype(jnp.float32)

    out_ref[0] = lax.dot_general(
        embeds_ref[...], onehot, (((0,), (0,)), ((), ())),
        preferred_element_type=jnp.float32)


def kernel(logits, key, embeds):
    b, d, t = logits.shape
    e = embeds.shape[1]
    t_blk = 2048

    grid_spec = pltpu.PrefetchScalarGridSpec(
        num_scalar_prefetch=1,
        grid=(b, t // t_blk),
        in_specs=[pl.BlockSpec((1, d, t_blk), lambda bi, ti, key: (bi, 0, ti)),
                  pl.BlockSpec((d, e), lambda bi, ti, key: (0, 0))],
        out_specs=pl.BlockSpec((1, e, t_blk), lambda bi, ti, key: (bi, 0, ti)),
    )
    out = pl.pallas_call(
        _sample_embed_kernel,
        out_shape=jax.ShapeDtypeStruct((b, e, t), jnp.float32),
        grid_spec=grid_spec,
        compiler_params=pltpu.CompilerParams(
            dimension_semantics=("parallel", "parallel"),
            vmem_limit_bytes=64 << 20),
    )(key, logits, embeds)
    return logits, out


# speedup vs baseline: 1.6405x; 1.0064x over previous
import numpy as np

import jax
import jax.numpy as jnp
from jax import lax
from jax.experimental import pallas as pl
from jax.experimental.pallas import tpu as pltpu

_TINY = np.float32(np.finfo(np.float32).tiny)
_CHUNK = 8


def _rotl(x, r):
    return lax.shift_left(x, jnp.uint32(r)) | lax.shift_right_logical(
        x, jnp.uint32(32 - r))


def _threefry2x32(k1, k2, x0, x1):
    ks = (k1, k2, k1 ^ k2 ^ jnp.uint32(0x1BD11BDA))
    rots = ((13, 15, 26, 6), (17, 29, 16, 24))
    sched = ((0, 1, 2), (1, 2, 0), (0, 0, 1), (1, 1, 2), (0, 2, 0))
    for i, (rset, ka, kb) in enumerate(sched):
        for r in rots[rset]:
            x0 = x0 + x1
            x1 = _rotl(x1, r)
            x1 = x0 ^ x1
        x0 = x0 + ks[ka]
        x1 = x1 + ks[kb] + jnp.uint32(i + 1)
    return x0, x1


def _sample_embed_kernel(key_ref, logits_ref, embeds_ref, out_ref):
    bi = pl.program_id(0)
    ti = pl.program_id(1)
    nb, d, tb = logits_ref.shape
    t_total = tb * pl.num_programs(1)
    k1 = key_ref[0]
    k2 = key_ref[1]

    d_iota = lax.broadcasted_iota(jnp.uint32, (_CHUNK, tb), 0)
    t_iota = lax.broadcasted_iota(jnp.uint32, (_CHUNK, tb), 1)
    inv = d_iota * jnp.uint32(t_total) + t_iota
    idx8 = lax.broadcasted_iota(jnp.int32, (_CHUNK, tb), 0)
    idx = lax.broadcasted_iota(jnp.int32, (d, tb), 0)

    for b_off in range(nb):
        base = (lax.convert_element_type(bi * nb + b_off, jnp.uint32)
                * jnp.uint32(d * t_total)
                + lax.convert_element_type(ti, jnp.uint32) * jnp.uint32(tb)
                + k2)

        m8 = None
        w8 = None
        for c in range(d // _CHUNK):
            x1 = inv + (base + jnp.uint32(c * _CHUNK * t_total))
            o0, o1 = _threefry2x32(k1, k2, k1 + jnp.zeros_like(x1), x1)
            bits = o0 ^ o1
            fb = (lax.shift_right_logical(bits, jnp.uint32(9))
                  | jnp.uint32(0x3F800000))
            f = lax.bitcast_convert_type(fb, jnp.float32) - jnp.float32(1.0)
            u = jnp.maximum(jnp.float32(_TINY), f + jnp.float32(_TINY))
            zc = (logits_ref[b_off, pl.ds(c * _CHUNK, _CHUNK), :]
                  - jnp.log(-jnp.log(u)))
            if m8 is None:
                m8, w8 = zc, idx8
            else:
                better = zc > m8
                w8 = jnp.where(better, idx8 + jnp.int32(c * _CHUNK), w8)
                m8 = jnp.maximum(m8, zc)

        zmax = jnp.max(m8, axis=0, keepdims=True)
        winner = jnp.min(jnp.where(m8 == zmax, w8, d), axis=0, keepdims=True)

        onehot = (idx == winner).astype(jnp.float32)

        out_ref[b_off] = lax.dot_general(
            embeds_ref[...], onehot, (((0,), (0,)), ((), ())),
            preferred_element_type=jnp.float32)


def kernel(logits, key, embeds):
    b, d, t = logits.shape
    e = embeds.shape[1]
    t_blk = next(c for c in (2048, 1024, 512, 256, 128, t) if t % c == 0)
    b_blk = 2 if b % 2 == 0 else 1

    grid_spec = pltpu.PrefetchScalarGridSpec(
        num_scalar_prefetch=1,
        grid=(b // b_blk, t // t_blk),
        in_specs=[pl.BlockSpec((b_blk, d, t_blk),
                               lambda bi, ti, key: (bi, 0, ti)),
                  pl.BlockSpec((d, e), lambda bi, ti, key: (0, 0))],
        out_specs=pl.BlockSpec((b_blk, e, t_blk),
                               lambda bi, ti, key: (bi, 0, ti)),
    )
    out = pl.pallas_call(
        _sample_embed_kernel,
        out_shape=jax.ShapeDtypeStruct((b, e, t), jnp.float32),
        grid_spec=grid_spec,
        compiler_params=pltpu.CompilerParams(
            dimension_semantics=("parallel", "parallel"),
            vmem_limit_bytes=64 << 20),
    )(key, logits, embeds)
    return logits, out


# speedup vs baseline: 1.6431x; 1.0016x over previous
import functools

import numpy as np

import jax
import jax.numpy as jnp
from jax import lax
from jax.experimental import pallas as pl
from jax.experimental.pallas import tpu as pltpu

_TINY = np.float32(np.finfo(np.float32).tiny)
_CHUNK = 8


def _rotl(x, r):
    return lax.shift_left(x, jnp.uint32(r)) | lax.shift_right_logical(
        x, jnp.uint32(32 - r))


def _threefry2x32(k1, k2, x0, x1):
    ks = (k1, k2, k1 ^ k2 ^ jnp.uint32(0x1BD11BDA))
    rots = ((13, 15, 26, 6), (17, 29, 16, 24))
    sched = ((0, 1, 2), (1, 2, 0), (0, 0, 1), (1, 1, 2), (0, 2, 0))
    for i, (rset, ka, kb) in enumerate(sched):
        for r in rots[rset]:
            x0 = x0 + x1
            x1 = _rotl(x1, r)
            x1 = x0 ^ x1
        x0 = x0 + ks[ka]
        x1 = x1 + ks[kb] + jnp.uint32(i + 1)
    return x0, x1


def _sample_embed_kernel(key_ref, logits_ref, embeds_ref, out_ref, *, t_full):
    bi = pl.program_id(0)
    ti = pl.program_id(1)
    nb, d, tb = logits_ref.shape
    k1 = key_ref[0]
    k2 = key_ref[1]

    d_iota = lax.broadcasted_iota(jnp.uint32, (_CHUNK, tb), 0)
    t_iota = lax.broadcasted_iota(jnp.uint32, (_CHUNK, tb), 1)
    inv = d_iota * jnp.uint32(t_full) + t_iota
    idx8 = lax.broadcasted_iota(jnp.int32, (_CHUNK, tb), 0)
    idx = lax.broadcasted_iota(jnp.int32, (d, tb), 0)

    for b_off in range(nb):
        base = (lax.convert_element_type(bi * nb + b_off, jnp.uint32)
                * jnp.uint32(d * t_full)
                + lax.convert_element_type(ti, jnp.uint32) * jnp.uint32(tb)
                + k2)

        m8 = None
        w8 = None
        for c in range(d // _CHUNK):
            x1 = inv + (base + jnp.uint32(c * _CHUNK * t_full))
            o0, o1 = _threefry2x32(k1, k2, k1, x1)
            bits = o0 ^ o1
            fb = (lax.shift_right_logical(bits, jnp.uint32(9))
                  | jnp.uint32(0x3F800000))
            f = lax.bitcast_convert_type(fb, jnp.float32) - jnp.float32(1.0)
            u = jnp.maximum(jnp.float32(_TINY), f + jnp.float32(_TINY))
            zc = (logits_ref[b_off, pl.ds(c * _CHUNK, _CHUNK), :]
                  - jnp.log(-jnp.log(u)))
            if m8 is None:
                m8 = zc
                w8 = jnp.zeros_like(idx8)
            else:
                w8 = jnp.where(zc > m8, jnp.int32(c), w8)
                m8 = jnp.maximum(m8, zc)

        winner8 = w8 * jnp.int32(_CHUNK) + idx8
        zmax = jnp.max(m8, axis=0, keepdims=True)
        winner = jnp.min(jnp.where(m8 == zmax, winner8, d),
                         axis=0, keepdims=True)

        onehot = (idx == winner).astype(jnp.float32)

        out_ref[b_off] = lax.dot_general(
            embeds_ref[...], onehot, (((0,), (0,)), ((), ())),
            preferred_element_type=jnp.float32)


def kernel(logits, key, embeds):
    b, d, t = logits.shape
    e = embeds.shape[1]
    t_blk = next(c for c in (2048, 1024, 512, 256, 128, t) if t % c == 0)
    b_blk = 2 if b % 2 == 0 else 1

    grid_spec = pltpu.PrefetchScalarGridSpec(
        num_scalar_prefetch=1,
        grid=(b // b_blk, t // t_blk),
        in_specs=[pl.BlockSpec((b_blk, d, t_blk),
                               lambda bi, ti, key: (bi, 0, ti)),
                  pl.BlockSpec((d, e), lambda bi, ti, key: (0, 0))],
        out_specs=pl.BlockSpec((b_blk, e, t_blk),
                               lambda bi, ti, key: (bi, 0, ti)),
    )
    out = pl.pallas_call(
        functools.partial(_sample_embed_kernel, t_full=t),
        out_shape=jax.ShapeDtypeStruct((b, e, t), jnp.float32),
        grid_spec=grid_spec,
        compiler_params=pltpu.CompilerParams(
            dimension_semantics=("parallel", "parallel"),
            vmem_limit_bytes=64 << 20),
    )(key, logits, embeds)
    return logits, out


# speedup vs baseline: 1.6985x; 1.0337x over previous
import functools

import numpy as np

import jax
import jax.numpy as jnp
from jax import lax
from jax.experimental import pallas as pl
from jax.experimental.pallas import tpu as pltpu

_TINY = np.float32(np.finfo(np.float32).tiny)
_CHUNK = 8


def _rotl(x, r):
    return lax.shift_left(x, jnp.uint32(r)) | lax.shift_right_logical(
        x, jnp.uint32(32 - r))


def _threefry2x32(k1, k2, x0, x1):
    ks = (k1, k2, k1 ^ k2 ^ jnp.uint32(0x1BD11BDA))
    rots = ((13, 15, 26, 6), (17, 29, 16, 24))
    sched = ((0, 1, 2), (1, 2, 0), (0, 0, 1), (1, 1, 2), (0, 2, 0))
    for i, (rset, ka, kb) in enumerate(sched):
        for r in rots[rset]:
            x0 = x0 + x1
            x1 = _rotl(x1, r)
            x1 = x0 ^ x1
        x0 = x0 + ks[ka]
        x1 = x1 + (ks[kb] + jnp.uint32(i + 1))
    return x0, x1


def _sample_embed_kernel(key_ref, logits_ref, embeds_ref, out_ref, *, t_full):
    bi = pl.program_id(0)
    ti = pl.program_id(1)
    nb, d, tb = logits_ref.shape
    k1 = key_ref[0]
    k2 = key_ref[1]

    d_iota = lax.broadcasted_iota(jnp.uint32, (_CHUNK, tb), 0)
    t_iota = lax.broadcasted_iota(jnp.uint32, (_CHUNK, tb), 1)
    inv = d_iota * jnp.uint32(t_full) + t_iota
    idx8 = lax.broadcasted_iota(jnp.int32, (_CHUNK, tb), 0)
    idx = lax.broadcasted_iota(jnp.int32, (d, tb), 0)

    for b_off in range(nb):
        base = (lax.convert_element_type(bi * nb + b_off, jnp.uint32)
                * jnp.uint32(d * t_full)
                + lax.convert_element_type(ti, jnp.uint32) * jnp.uint32(tb)
                + k2)

        m8 = None
        w8 = None
        for c in range(d // _CHUNK):
            x1 = inv + (base + jnp.uint32(c * _CHUNK * t_full))
            o0, o1 = _threefry2x32(k1, k2, k1, x1)
            bits = o0 ^ o1
            fb = (lax.shift_right_logical(bits, jnp.uint32(9))
                  | jnp.uint32(0x3F800000))
            f = lax.bitcast_convert_type(fb, jnp.float32) - jnp.float32(1.0)
            u = jnp.maximum(jnp.float32(_TINY), f + jnp.float32(_TINY))
            zc = (logits_ref[b_off, pl.ds(c * _CHUNK, _CHUNK), :]
                  - jnp.log(-jnp.log(u)))
            if m8 is None:
                m8 = zc
                w8 = jnp.zeros_like(idx8)
            else:
                w8 = jnp.where(zc > m8, jnp.int32(c), w8)
                m8 = jnp.maximum(m8, zc)

        winner8 = w8 * jnp.int32(_CHUNK) + idx8
        zmax = jnp.max(m8, axis=0, keepdims=True)
        winner = jnp.min(jnp.where(m8 == zmax, winner8, d),
                         axis=0, keepdims=True)

        onehot = (idx == winner).astype(jnp.float32)

        out_ref[b_off] = lax.dot_general(
            embeds_ref[...], onehot, (((0,), (0,)), ((), ())),
            preferred_element_type=jnp.float32)


def kernel(logits, key, embeds):
    b, d, t = logits.shape
    e = embeds.shape[1]
    t_blk = next(c for c in (2048, 1024, 512, 256, 128, t) if t % c == 0)
    b_blk = 2 if b % 2 == 0 else 1

    grid_spec = pltpu.PrefetchScalarGridSpec(
        num_scalar_prefetch=1,
        grid=(b // b_blk, t // t_blk),
        in_specs=[pl.BlockSpec((b_blk, d, t_blk),
                               lambda bi, ti, key: (bi, 0, ti)),
                  pl.BlockSpec((d, e), lambda bi, ti, key: (0, 0))],
        out_specs=pl.BlockSpec((b_blk, e, t_blk),
                               lambda bi, ti, key: (bi, 0, ti)),
    )
    out = pl.pallas_call(
        functools.partial(_sample_embed_kernel, t_full=t),
        out_shape=jax.ShapeDtypeStruct((b, e, t), jnp.float32),
        grid_spec=grid_spec,
        compiler_params=pltpu.CompilerParams(
            dimension_semantics=("parallel", "parallel"),
            vmem_limit_bytes=64 << 20),
    )(key, logits, embeds)
    return logits, out
